# Initial kernel scaffold; baseline (speedup 1.0000x reference)
import jax
import jax.numpy as jnp
from jax.experimental import pallas as pl


def kernel(x, edge_index, edge_attr, params):
    raise NotImplementedError("write your pallas kernel here")


# speedup vs baseline: 3.1689x; 3.1689x over previous
import functools

import jax
import jax.numpy as jnp
from jax import lax
from jax.experimental import pallas as pl
from jax.experimental.pallas import tpu as pltpu
from jax.experimental.pallas import tpu_sc as plsc

N = 10000
E = 160000
EDGE_DIM = 16
NP = 10240
NT = 10368
E_PAD = 163840
NC, NS = 2, 16
NW = NC * NS
CH = 128
CPW = E_PAD // (NW * CH)
RPT = NT // NS
WB = 24
ZR = 8


def _mesh():
    return plsc.VectorSubcoreMesh(
        core_axis_name="c", subcore_axis_name="s",
        num_cores=NC, num_subcores=NS)


@functools.cache
def _sc_gather_kernel():
    return pl.kernel(
        _sc_gather_body,
        out_type=jax.ShapeDtypeStruct((E_PAD, 256), jnp.float32),
        mesh=_mesh(),
        scratch_types=[
            pltpu.VMEM((CH,), jnp.int32),
            pltpu.VMEM((CH, 256), jnp.float32),
            pltpu.SemaphoreType.DMA,
        ],
    )


def _sc_gather(table, idx):
    return _sc_gather_kernel()(table, idx)


def _sc_gather_body(table_hbm, idx_hbm, out_hbm, idx_v, rows_v, sem):
    wid = lax.axis_index("s") * NC + lax.axis_index("c")

    def body(j, carry):
        base = (wid * CPW + j) * CH
        pltpu.sync_copy(idx_hbm.at[pl.ds(base, CH)], idx_v)
        pltpu.async_copy(table_hbm.at[idx_v], rows_v, sem).wait()
        pltpu.sync_copy(rows_v, out_hbm.at[pl.ds(base, CH)])
        return carry

    lax.fori_loop(0, CPW, body, 0)


@functools.cache
def _sc_scatter_kernel():
    return pl.kernel(
        _sc_scatter_body,
        out_type=jax.ShapeDtypeStruct((NC, NT, 128), jnp.float32),
        mesh=_mesh(),
        scratch_types=[
            pltpu.VMEM((CH,), jnp.int32),
            pltpu.VMEM((CH,), jnp.int32),
            pltpu.VMEM((CH, 128), jnp.float32),
            pltpu.VMEM((CH, 128), jnp.float32),
            pltpu.VMEM((ZR, 128), jnp.float32),
            pltpu.VMEM_SHARED((NT, 128), jnp.float32),
            pltpu.SemaphoreType.DMA,
        ],
    )


def _sc_scatter(vm, vd, idx, idx2):
    return _sc_scatter_kernel()(vm, vd, idx, idx2)


def _sc_scatter_body(vm_hbm, vd_hbm, idx_hbm, idx2_hbm, out_hbm,
                     idx_v, idx2_v, val_v, vd_v, z_v, acc_sh, sem):
    cid = lax.axis_index("c")
    sid = lax.axis_index("s")
    wid = sid * NC + cid

    zeros16 = jnp.zeros((16,), jnp.float32)
    for r in range(ZR):
        for c in range(8):
            z_v[r, pl.ds(c * 16, 16)] = zeros16

    def zero_body(k, carry):
        pltpu.sync_copy(z_v, acc_sh.at[pl.ds(sid * RPT + k * ZR, ZR)])
        return carry

    lax.fori_loop(0, RPT // ZR, zero_body, 0)
    plsc.subcore_barrier()

    def body(j, carry):
        base = (wid * CPW + j) * CH
        pltpu.sync_copy(idx_hbm.at[pl.ds(base, CH)], idx_v)
        pltpu.sync_copy(idx2_hbm.at[pl.ds(base, CH)], idx2_v)
        pltpu.sync_copy(vm_hbm.at[pl.ds(base, CH)], val_v)
        pltpu.sync_copy(vd_hbm.at[pl.ds(base, CH)], vd_v)
        pltpu.sync_copy(val_v, acc_sh.at[idx_v], add=True)
        pltpu.sync_copy(vd_v, acc_sh.at[idx2_v], add=True)
        return carry

    lax.fori_loop(0, CPW, body, 0)
    plsc.subcore_barrier()

    def wb_body(k, carry):
        row = sid * RPT + k * WB
        pltpu.sync_copy(acc_sh.at[pl.ds(row, WB)], val_v.at[pl.ds(0, WB)])
        pltpu.sync_copy(val_v.at[pl.ds(0, WB)], out_hbm.at[cid].at[pl.ds(row, WB)])
        return carry

    lax.fori_loop(0, RPT // WB, wb_body, 0)


def _dense1_body(x_ref, wcat_ref, u_ref, bcat_ref, gq_ref, r_ref):
    xb = x_ref[...]
    ti = jnp.clip(xb[:, 0:1].astype(jnp.int32), 0, 1).astype(jnp.float32)
    gqr = jnp.dot(xb, wcat_ref[...], preferred_element_type=jnp.float32)
    usel = u_ref[0:1, :] + ti * (u_ref[1:2, :] - u_ref[0:1, :])
    gqr = gqr + usel + bcat_ref[...]
    gq_ref[...] = gqr[:, :256]
    r_ref[...] = gqr[:, 256:]


def _edge_mlp_body(gq_ref, ea_ref, dmod_ref, wec_ref, wh1_ref, bh1_ref,
                   wh2_ref, bh2_ref, w2_ref, b2_ref, vm_ref, vd_ref):
    gq = gq_ref[...]
    eaa = jnp.dot(ea_ref[...], wec_ref[...], preferred_element_type=jnp.float32)
    g = gq[:, :128] + eaa[:, :128]
    m = gq[:, 128:] + eaa[:, 128:]
    t = jnp.tanh(g)
    t = jnp.tanh(jnp.dot(t, wh1_ref[...], preferred_element_type=jnp.float32)
                 + bh1_ref[...])
    t = jnp.tanh(jnp.dot(t, wh2_ref[...], preferred_element_type=jnp.float32)
                 + bh2_ref[...])
    gate = jnp.sum(t * w2_ref[...], axis=1, keepdims=True) + b2_ref[...]
    ex = jnp.exp(gate)
    vm_ref[...] = ex * m
    onehot = (lax.broadcasted_iota(jnp.int32, (1, 128), 1) == dmod_ref[...])
    vd_ref[...] = ex * onehot.astype(jnp.float32)


def _combine2_body(p_ref, den_ref, r1_ref, bl1_ref, wcat2_ref, bcat2_ref,
                   gq2_ref, r2_ref):
    s = p_ref[0] + p_ref[1]
    aggr = s / (den_ref[...] + 1e-16)
    h2 = jnp.maximum(aggr + bl1_ref[...] + r1_ref[...], 0.0)
    gqr = jnp.dot(h2, wcat2_ref[...], preferred_element_type=jnp.float32)
    gqr = gqr + bcat2_ref[...]
    gq2_ref[...] = gqr[:, :256]
    r2_ref[...] = gqr[:, 256:]


def _final_body(p_ref, den_ref, r2_ref, bl2_ref, o_ref):
    s = p_ref[0] + p_ref[1]
    aggr = s / (den_ref[...] + 1e-16)
    o_ref[...] = aggr + bl2_ref[...] + r2_ref[...]


_NB = 1000
_EB = 640


def _rows_spec(blk, width):
    return pl.BlockSpec((blk, width), lambda i: (i, 0))


def _bcast_spec(shape):
    return pl.BlockSpec(shape, lambda i: tuple(0 for _ in shape))


def _dense(x_in, wcat, u, bcat):
    return pl.pallas_call(
        _dense1_body,
        grid=(N // _NB,),
        in_specs=[_rows_spec(_NB, 128), _bcast_spec((128, 384)),
                  _bcast_spec((2, 384)), _bcast_spec((1, 384))],
        out_specs=[_rows_spec(_NB, 256), _rows_spec(_NB, 128)],
        out_shape=[jax.ShapeDtypeStruct((N, 256), jnp.float32),
                   jax.ShapeDtypeStruct((N, 128), jnp.float32)],
    )(x_in, wcat, u, bcat)


def _edge_mlp(gqg, eap, dmod, wec, wh1, bh1, wh2, bh2, w2, b2):
    return pl.pallas_call(
        _edge_mlp_body,
        grid=(E_PAD // _EB,),
        in_specs=[_rows_spec(_EB, 256), _rows_spec(_EB, EDGE_DIM),
                  _rows_spec(_EB, 1),
                  _bcast_spec((EDGE_DIM, 256)), _bcast_spec((128, 128)),
                  _bcast_spec((1, 128)), _bcast_spec((128, 128)),
                  _bcast_spec((1, 128)), _bcast_spec((1, 128)),
                  _bcast_spec((1, 1))],
        out_specs=[_rows_spec(_EB, 128), _rows_spec(_EB, 128)],
        out_shape=[jax.ShapeDtypeStruct((E_PAD, 128), jnp.float32),
                   jax.ShapeDtypeStruct((E_PAD, 128), jnp.float32)],
    )(gqg, eap, dmod, wec, wh1, bh1, wh2, bh2, w2, b2)


def _combine2(p, den, r1, bl1, wcat2, bcat2):
    return pl.pallas_call(
        _combine2_body,
        grid=(N // _NB,),
        in_specs=[pl.BlockSpec((NC, _NB, 128), lambda i: (0, i, 0)),
                  _rows_spec(_NB, 1),
                  _rows_spec(_NB, 128), _bcast_spec((1, 128)),
                  _bcast_spec((128, 384)), _bcast_spec((1, 384))],
        out_specs=[_rows_spec(_NB, 256), _rows_spec(_NB, 128)],
        out_shape=[jax.ShapeDtypeStruct((N, 256), jnp.float32),
                   jax.ShapeDtypeStruct((N, 128), jnp.float32)],
    )(p, den, r1, bl1, wcat2, bcat2)


def _final(p, den, r2, bl2):
    return pl.pallas_call(
        _final_body,
        grid=(N // _NB,),
        in_specs=[pl.BlockSpec((NC, _NB, 128), lambda i: (0, i, 0)),
                  _rows_spec(_NB, 1),
                  _rows_spec(_NB, 128), _bcast_spec((1, 128))],
        out_specs=_rows_spec(_NB, 128),
        out_shape=jax.ShapeDtypeStruct((N, 128), jnp.float32),
    )(p, den, r2, bl2)


def kernel(x, edge_index, edge_attr, params):
    p = params
    f32 = jnp.float32

    zrow = jnp.zeros((1, 128), f32)
    W1, Wl, Wr = p['l1_gate_W1'], p['l1_Wl'], p['l1_Wr']
    wcat1 = jnp.concatenate([
        jnp.concatenate([zrow, W1[:127]], 0),
        jnp.concatenate([zrow, Wl[:127]], 0),
        jnp.concatenate([zrow, Wr[:127]], 0)], 1)
    u1 = jnp.concatenate([p['embed'] @ W1[127:129],
                          p['embed'] @ Wl[127:129],
                          p['embed'] @ Wr[127:129]], 1)
    bcat1 = jnp.concatenate([p['l1_gate_b1'], jnp.zeros((256,), f32)])[None]
    wec1 = jnp.concatenate([W1[129:], Wl[129:]], 1)

    z16 = jnp.zeros((16, 128), f32)
    W1_2, Wl_2, Wr_2 = p['l2_gate_W1'], p['l2_Wl'], p['l2_Wr']
    wcat2 = jnp.concatenate([
        jnp.concatenate([W1_2[:112], z16], 0),
        jnp.concatenate([Wl_2[:112], z16], 0),
        Wr_2], 1)
    bcat2 = jnp.concatenate([p['l2_gate_b1'], jnp.zeros((256,), f32)])[None]
    wec2 = jnp.concatenate([W1_2[112:], Wl_2[112:]], 1)

    src = edge_index[0]
    dst = edge_index[1]
    pad = E_PAD - E
    srcp = jnp.concatenate([src, jnp.zeros((pad,), jnp.int32)])
    dstp = jnp.concatenate([dst, jnp.full((pad,), N, jnp.int32)])
    dst2p = NP + dstp // 128
    dmod = (dstp % 128).reshape(E_PAD, 1)
    eap = jnp.concatenate([edge_attr, jnp.zeros((pad, EDGE_DIM), f32)], 0)

    def layer(gq, lname, wec):
        gqg = _sc_gather(gq, srcp)
        vm, vd = _edge_mlp(gqg, eap, dmod, wec,
                           p[lname + '_gate_Wh1'], p[lname + '_gate_bh1'][None],
                           p[lname + '_gate_Wh2'], p[lname + '_gate_bh2'][None],
                           p[lname + '_gate_W2'].T, p[lname + '_gate_b2'][None])
        acc = _sc_scatter(vm, vd, dstp, dst2p)
        den = acc[:, NP:NP + (N + 127) // 128, :].sum(0).reshape(-1)[:N, None]
        return acc, den

    gq1, r1 = _dense(x, wcat1, u1, bcat1)
    p1, den1 = layer(gq1, 'l1', wec1)
    gq2, r2 = _combine2(p1, den1, r1, p['l1_bl'][None], wcat2, bcat2)
    p2, den2 = layer(gq2, 'l2', wec2)
    return _final(p2, den2, r2, p['l2_bl'][None])


# speedup vs baseline: 3.3438x; 1.0552x over previous
import functools

import jax
import jax.numpy as jnp
from jax import lax
from jax.experimental import pallas as pl
from jax.experimental.pallas import tpu as pltpu
from jax.experimental.pallas import tpu_sc as plsc

N = 10000
E = 160000
EDGE_DIM = 16
NP = 10240
NT = 10368
E_PAD = 163840
NC, NS = 2, 16
NW = NC * NS
CH = 128
CPW = E_PAD // (NW * CH)
RPT = NT // NS
WB = 24
ZR = 8


def _mesh():
    return plsc.VectorSubcoreMesh(
        core_axis_name="c", subcore_axis_name="s",
        num_cores=NC, num_subcores=NS)


@functools.cache
def _sc_gather_kernel():
    return pl.kernel(
        _sc_gather_body,
        out_type=jax.ShapeDtypeStruct((E_PAD, 256), jnp.float32),
        mesh=_mesh(),
        scratch_types=[
            pltpu.VMEM((CPW, CH), jnp.int32),
            pltpu.VMEM((CH, 256), jnp.float32),
            pltpu.VMEM((CH, 256), jnp.float32),
            pltpu.VMEM((CH, 256), jnp.float32),
            pltpu.SemaphoreType.DMA,
            pltpu.SemaphoreType.DMA,
            pltpu.SemaphoreType.DMA,
            pltpu.SemaphoreType.DMA,
            pltpu.SemaphoreType.DMA,
            pltpu.SemaphoreType.DMA,
        ],
    )


def _sc_gather(table, idx2d):
    return _sc_gather_kernel()(table, idx2d)


def _sc_gather_body(table_hbm, idx_hbm, out_hbm, idx_all, r0, r1, r2,
                    g0, g1, g2, s0, s1, s2):
    wid = lax.axis_index("s") * NC + lax.axis_index("c")
    base_w = wid * CPW
    pltpu.sync_copy(idx_hbm.at[pl.ds(base_w, CPW)], idx_all)
    rows = (r0, r1, r2)
    gsem = (g0, g1, g2)
    ssem = (s0, s1, s2)

    def g_copy(j, b):
        return pltpu.make_async_copy(table_hbm.at[idx_all.at[j]], rows[b],
                                     gsem[b])

    def s_copy(j, b):
        return pltpu.make_async_copy(
            rows[b], out_hbm.at[pl.ds((base_w + j) * CH, CH)], ssem[b])

    g_copy(0, 0).start()
    g_copy(1, 1).start()
    g_copy(0, 0).wait()
    s_copy(0, 0).start()
    g_copy(2, 2).start()

    def group(gi, carry):
        j0 = 1 + 3 * gi
        g_copy(j0, 1).wait()
        s_copy(j0, 1).start()
        s_copy(j0 - 1, 0).wait()
        g_copy(j0 + 2, 0).start()
        g_copy(j0 + 1, 2).wait()
        s_copy(j0 + 1, 2).start()
        s_copy(j0, 1).wait()
        g_copy(j0 + 3, 1).start()
        g_copy(j0 + 2, 0).wait()
        s_copy(j0 + 2, 0).start()
        s_copy(j0 + 1, 2).wait()
        g_copy(j0 + 4, 2).start()
        return carry

    lax.fori_loop(0, (CPW - 4) // 3, group, 0)

    g_copy(CPW - 3, 1).wait()
    s_copy(CPW - 3, 1).start()
    s_copy(CPW - 4, 0).wait()
    g_copy(CPW - 1, 0).start()
    g_copy(CPW - 2, 2).wait()
    s_copy(CPW - 2, 2).start()
    g_copy(CPW - 1, 0).wait()
    s_copy(CPW - 1, 0).start()
    s_copy(CPW - 3, 1).wait()
    s_copy(CPW - 2, 2).wait()
    s_copy(CPW - 1, 0).wait()


@functools.cache
def _sc_scatter_kernel():
    return pl.kernel(
        _sc_scatter_body,
        out_type=jax.ShapeDtypeStruct((NC, NT, 128), jnp.float32),
        mesh=_mesh(),
        scratch_types=[
            pltpu.VMEM((CH,), jnp.int32),
            pltpu.VMEM((CH,), jnp.int32),
            pltpu.VMEM((CH, 128), jnp.float32),
            pltpu.VMEM((CH, 128), jnp.float32),
            pltpu.VMEM((ZR, 128), jnp.float32),
            pltpu.VMEM_SHARED((NT, 128), jnp.float32),
            pltpu.SemaphoreType.DMA,
        ],
    )


def _sc_scatter(vm, vd, idx, idx2):
    return _sc_scatter_kernel()(vm, vd, idx, idx2)


def _sc_scatter_body(vm_hbm, vd_hbm, idx_hbm, idx2_hbm, out_hbm,
                     idx_v, idx2_v, val_v, vd_v, z_v, acc_sh, sem):
    cid = lax.axis_index("c")
    sid = lax.axis_index("s")
    wid = sid * NC + cid

    zeros16 = jnp.zeros((16,), jnp.float32)
    for r in range(ZR):
        for c in range(8):
            z_v[r, pl.ds(c * 16, 16)] = zeros16

    def zero_body(k, carry):
        pltpu.sync_copy(z_v, acc_sh.at[pl.ds(sid * RPT + k * ZR, ZR)])
        return carry

    lax.fori_loop(0, RPT // ZR, zero_body, 0)
    plsc.subcore_barrier()

    def body(j, carry):
        base = (wid * CPW + j) * CH
        pltpu.sync_copy(idx_hbm.at[pl.ds(base, CH)], idx_v)
        pltpu.sync_copy(idx2_hbm.at[pl.ds(base, CH)], idx2_v)
        pltpu.sync_copy(vm_hbm.at[pl.ds(base, CH)], val_v)
        pltpu.sync_copy(vd_hbm.at[pl.ds(base, CH)], vd_v)
        pltpu.sync_copy(val_v, acc_sh.at[idx_v], add=True)
        pltpu.sync_copy(vd_v, acc_sh.at[idx2_v], add=True)
        return carry

    lax.fori_loop(0, CPW, body, 0)
    plsc.subcore_barrier()

    def wb_body(k, carry):
        row = sid * RPT + k * WB
        pltpu.sync_copy(acc_sh.at[pl.ds(row, WB)], val_v.at[pl.ds(0, WB)])
        pltpu.sync_copy(val_v.at[pl.ds(0, WB)], out_hbm.at[cid].at[pl.ds(row, WB)])
        return carry

    lax.fori_loop(0, RPT // WB, wb_body, 0)


def _dense1_body(x_ref, wcat_ref, u_ref, bcat_ref, gq_ref, r_ref):
    xb = x_ref[...]
    ti = jnp.clip(xb[:, 0:1].astype(jnp.int32), 0, 1).astype(jnp.float32)
    gqr = jnp.dot(xb, wcat_ref[...], preferred_element_type=jnp.float32)
    usel = u_ref[0:1, :] + ti * (u_ref[1:2, :] - u_ref[0:1, :])
    gqr = gqr + usel + bcat_ref[...]
    gq_ref[...] = gqr[:, :256]
    r_ref[...] = gqr[:, 256:]


def _edge_mlp_body(gq_ref, ea_ref, dmod_ref, wec_ref, wh1_ref, bh1_ref,
                   wh2_ref, bh2_ref, w2_ref, b2_ref, vm_ref, vd_ref):
    gq = gq_ref[...]
    eaa = jnp.dot(ea_ref[...], wec_ref[...], preferred_element_type=jnp.float32)
    g = gq[:, :128] + eaa[:, :128]
    m = gq[:, 128:] + eaa[:, 128:]
    t = jnp.tanh(g)
    t = jnp.tanh(jnp.dot(t, wh1_ref[...], preferred_element_type=jnp.float32)
                 + bh1_ref[...])
    t = jnp.tanh(jnp.dot(t, wh2_ref[...], preferred_element_type=jnp.float32)
                 + bh2_ref[...])
    gate = jnp.sum(t * w2_ref[...], axis=1, keepdims=True) + b2_ref[...]
    ex = jnp.exp(gate)
    vm_ref[...] = ex * m
    onehot = (lax.broadcasted_iota(jnp.int32, (1, 128), 1) == dmod_ref[...])
    vd_ref[...] = ex * onehot.astype(jnp.float32)


def _combine2_body(p_ref, den_ref, r1_ref, bl1_ref, wcat2_ref, bcat2_ref,
                   gq2_ref, r2_ref):
    s = p_ref[0] + p_ref[1]
    aggr = s / (den_ref[...] + 1e-16)
    h2 = jnp.maximum(aggr + bl1_ref[...] + r1_ref[...], 0.0)
    gqr = jnp.dot(h2, wcat2_ref[...], preferred_element_type=jnp.float32)
    gqr = gqr + bcat2_ref[...]
    gq2_ref[...] = gqr[:, :256]
    r2_ref[...] = gqr[:, 256:]


def _final_body(p_ref, den_ref, r2_ref, bl2_ref, o_ref):
    s = p_ref[0] + p_ref[1]
    aggr = s / (den_ref[...] + 1e-16)
    o_ref[...] = aggr + bl2_ref[...] + r2_ref[...]


_NB = 1000
_EB = 640


def _rows_spec(blk, width):
    return pl.BlockSpec((blk, width), lambda i: (i, 0))


def _bcast_spec(shape):
    return pl.BlockSpec(shape, lambda i: tuple(0 for _ in shape))


def _dense(x_in, wcat, u, bcat):
    return pl.pallas_call(
        _dense1_body,
        grid=(N // _NB,),
        in_specs=[_rows_spec(_NB, 128), _bcast_spec((128, 384)),
                  _bcast_spec((2, 384)), _bcast_spec((1, 384))],
        out_specs=[_rows_spec(_NB, 256), _rows_spec(_NB, 128)],
        out_shape=[jax.ShapeDtypeStruct((N, 256), jnp.float32),
                   jax.ShapeDtypeStruct((N, 128), jnp.float32)],
    )(x_in, wcat, u, bcat)


def _edge_mlp(gqg, eap, dmod, wec, wh1, bh1, wh2, bh2, w2, b2):
    return pl.pallas_call(
        _edge_mlp_body,
        grid=(E_PAD // _EB,),
        in_specs=[_rows_spec(_EB, 256), _rows_spec(_EB, EDGE_DIM),
                  _rows_spec(_EB, 1),
                  _bcast_spec((EDGE_DIM, 256)), _bcast_spec((128, 128)),
                  _bcast_spec((1, 128)), _bcast_spec((128, 128)),
                  _bcast_spec((1, 128)), _bcast_spec((1, 128)),
                  _bcast_spec((1, 1))],
        out_specs=[_rows_spec(_EB, 128), _rows_spec(_EB, 128)],
        out_shape=[jax.ShapeDtypeStruct((E_PAD, 128), jnp.float32),
                   jax.ShapeDtypeStruct((E_PAD, 128), jnp.float32)],
    )(gqg, eap, dmod, wec, wh1, bh1, wh2, bh2, w2, b2)


def _combine2(p, den, r1, bl1, wcat2, bcat2):
    return pl.pallas_call(
        _combine2_body,
        grid=(N // _NB,),
        in_specs=[pl.BlockSpec((NC, _NB, 128), lambda i: (0, i, 0)),
                  _rows_spec(_NB, 1),
                  _rows_spec(_NB, 128), _bcast_spec((1, 128)),
                  _bcast_spec((128, 384)), _bcast_spec((1, 384))],
        out_specs=[_rows_spec(_NB, 256), _rows_spec(_NB, 128)],
        out_shape=[jax.ShapeDtypeStruct((N, 256), jnp.float32),
                   jax.ShapeDtypeStruct((N, 128), jnp.float32)],
    )(p, den, r1, bl1, wcat2, bcat2)


def _final(p, den, r2, bl2):
    return pl.pallas_call(
        _final_body,
        grid=(N // _NB,),
        in_specs=[pl.BlockSpec((NC, _NB, 128), lambda i: (0, i, 0)),
                  _rows_spec(_NB, 1),
                  _rows_spec(_NB, 128), _bcast_spec((1, 128))],
        out_specs=_rows_spec(_NB, 128),
        out_shape=jax.ShapeDtypeStruct((N, 128), jnp.float32),
    )(p, den, r2, bl2)


def kernel(x, edge_index, edge_attr, params):
    p = params
    f32 = jnp.float32

    zrow = jnp.zeros((1, 128), f32)
    W1, Wl, Wr = p['l1_gate_W1'], p['l1_Wl'], p['l1_Wr']
    wcat1 = jnp.concatenate([
        jnp.concatenate([zrow, W1[:127]], 0),
        jnp.concatenate([zrow, Wl[:127]], 0),
        jnp.concatenate([zrow, Wr[:127]], 0)], 1)
    u1 = jnp.concatenate([p['embed'] @ W1[127:129],
                          p['embed'] @ Wl[127:129],
                          p['embed'] @ Wr[127:129]], 1)
    bcat1 = jnp.concatenate([p['l1_gate_b1'], jnp.zeros((256,), f32)])[None]
    wec1 = jnp.concatenate([W1[129:], Wl[129:]], 1)

    z16 = jnp.zeros((16, 128), f32)
    W1_2, Wl_2, Wr_2 = p['l2_gate_W1'], p['l2_Wl'], p['l2_Wr']
    wcat2 = jnp.concatenate([
        jnp.concatenate([W1_2[:112], z16], 0),
        jnp.concatenate([Wl_2[:112], z16], 0),
        Wr_2], 1)
    bcat2 = jnp.concatenate([p['l2_gate_b1'], jnp.zeros((256,), f32)])[None]
    wec2 = jnp.concatenate([W1_2[112:], Wl_2[112:]], 1)

    src = edge_index[0]
    dst = edge_index[1]
    pad = E_PAD - E
    srcp = jnp.concatenate([src, jnp.zeros((pad,), jnp.int32)])
    srcp2d = srcp.reshape(E_PAD // CH, CH)
    dstp = jnp.concatenate([dst, jnp.full((pad,), N, jnp.int32)])
    dst2p = NP + dstp // 128
    dmod = (dstp % 128).reshape(E_PAD, 1)
    eap = jnp.concatenate([edge_attr, jnp.zeros((pad, EDGE_DIM), f32)], 0)

    def layer(gq, lname, wec):
        gqg = _sc_gather(gq, srcp2d)
        vm, vd = _edge_mlp(gqg, eap, dmod, wec,
                           p[lname + '_gate_Wh1'], p[lname + '_gate_bh1'][None],
                           p[lname + '_gate_Wh2'], p[lname + '_gate_bh2'][None],
                           p[lname + '_gate_W2'].T, p[lname + '_gate_b2'][None])
        acc = _sc_scatter(vm, vd, dstp, dst2p)
        den = acc[:, NP:NP + (N + 127) // 128, :].sum(0).reshape(-1)[:N, None]
        return acc, den

    gq1, r1 = _dense(x, wcat1, u1, bcat1)
    p1, den1 = layer(gq1, 'l1', wec1)
    gq2, r2 = _combine2(p1, den1, r1, p['l1_bl'][None], wcat2, bcat2)
    p2, den2 = layer(gq2, 'l2', wec2)
    return _final(p2, den2, r2, p['l2_bl'][None])


# speedup vs baseline: 3.6373x; 1.0878x over previous
import functools

import jax
import jax.numpy as jnp
from jax import lax
from jax.experimental import pallas as pl
from jax.experimental.pallas import tpu as pltpu
from jax.experimental.pallas import tpu_sc as plsc

N = 10000
E = 160000
EDGE_DIM = 16
NP = 10240
DB = 10008
NT = 10112
E_PAD = 163840
NC, NS = 2, 16
NW = NC * NS
CH = 128
CPW = E_PAD // (NW * CH)
SCH = 64
SCPW = E_PAD // (NW * SCH)
RPT = NT // NS
WB = 8
ZR = 8


def _mesh():
    return plsc.VectorSubcoreMesh(
        core_axis_name="c", subcore_axis_name="s",
        num_cores=NC, num_subcores=NS)


@functools.cache
def _sc_gather_kernel():
    return pl.kernel(
        _sc_gather_body,
        out_type=jax.ShapeDtypeStruct((E_PAD, 256), jnp.float32),
        mesh=_mesh(),
        scratch_types=[
            pltpu.VMEM((CPW, CH), jnp.int32),
            pltpu.VMEM((CH, 256), jnp.float32),
            pltpu.VMEM((CH, 256), jnp.float32),
            pltpu.VMEM((CH, 256), jnp.float32),
            pltpu.SemaphoreType.DMA,
            pltpu.SemaphoreType.DMA,
            pltpu.SemaphoreType.DMA,
            pltpu.SemaphoreType.DMA,
            pltpu.SemaphoreType.DMA,
            pltpu.SemaphoreType.DMA,
        ],
    )


def _sc_gather(table, idx2d):
    return _sc_gather_kernel()(table, idx2d)


def _sc_gather_body(table_hbm, idx_hbm, out_hbm, idx_all, r0, r1, r2,
                    g0, g1, g2, s0, s1, s2):
    wid = lax.axis_index("s") * NC + lax.axis_index("c")
    base_w = wid * CPW
    pltpu.sync_copy(idx_hbm.at[pl.ds(base_w, CPW)], idx_all)
    rows = (r0, r1, r2)
    gsem = (g0, g1, g2)
    ssem = (s0, s1, s2)

    def g_copy(j, b):
        return pltpu.make_async_copy(table_hbm.at[idx_all.at[j]], rows[b],
                                     gsem[b])

    def s_copy(j, b):
        return pltpu.make_async_copy(
            rows[b], out_hbm.at[pl.ds((base_w + j) * CH, CH)], ssem[b])

    g_copy(0, 0).start()
    g_copy(1, 1).start()
    g_copy(0, 0).wait()
    s_copy(0, 0).start()
    g_copy(2, 2).start()

    def group(gi, carry):
        j0 = 1 + 3 * gi
        g_copy(j0, 1).wait()
        s_copy(j0, 1).start()
        s_copy(j0 - 1, 0).wait()
        g_copy(j0 + 2, 0).start()
        g_copy(j0 + 1, 2).wait()
        s_copy(j0 + 1, 2).start()
        s_copy(j0, 1).wait()
        g_copy(j0 + 3, 1).start()
        g_copy(j0 + 2, 0).wait()
        s_copy(j0 + 2, 0).start()
        s_copy(j0 + 1, 2).wait()
        g_copy(j0 + 4, 2).start()
        return carry

    lax.fori_loop(0, (CPW - 4) // 3, group, 0)

    g_copy(CPW - 3, 1).wait()
    s_copy(CPW - 3, 1).start()
    s_copy(CPW - 4, 0).wait()
    g_copy(CPW - 1, 0).start()
    g_copy(CPW - 2, 2).wait()
    s_copy(CPW - 2, 2).start()
    g_copy(CPW - 1, 0).wait()
    s_copy(CPW - 1, 0).start()
    s_copy(CPW - 3, 1).wait()
    s_copy(CPW - 2, 2).wait()
    s_copy(CPW - 1, 0).wait()


@functools.cache
def _sc_scatter_kernel():
    return pl.kernel(
        _sc_scatter_body,
        out_type=jax.ShapeDtypeStruct((NC, NT, 128), jnp.float32),
        mesh=_mesh(),
        scratch_types=[
            pltpu.VMEM((2, SCH), jnp.int32),
            pltpu.VMEM((2, SCH), jnp.int32),
            pltpu.VMEM((SCH, 128), jnp.float32),
            pltpu.VMEM((SCH, 128), jnp.float32),
            pltpu.VMEM((SCH, 128), jnp.float32),
            pltpu.VMEM((SCH, 128), jnp.float32),
            pltpu.VMEM((ZR, 128), jnp.float32),
            pltpu.VMEM_SHARED((NT, 128), jnp.float32),
            pltpu.SemaphoreType.DMA,
            pltpu.SemaphoreType.DMA,
            pltpu.SemaphoreType.DMA,
            pltpu.SemaphoreType.DMA,
            pltpu.SemaphoreType.DMA,
            pltpu.SemaphoreType.DMA,
            pltpu.SemaphoreType.DMA,
            pltpu.SemaphoreType.DMA,
            pltpu.SemaphoreType.DMA,
            pltpu.SemaphoreType.DMA,
            pltpu.SemaphoreType.DMA,
            pltpu.SemaphoreType.DMA,
        ],
    )


def _sc_scatter(vm, vd, idx2d, idx2_2d):
    return _sc_scatter_kernel()(vm, vd, idx2d, idx2_2d)


def _sc_scatter_body(vm_hbm, vd_hbm, idx_hbm, idx2_hbm, out_hbm,
                     idx_pp, idx2_pp, vm0, vm1, vd0, vd1,
                     z_v, acc_sh,
                     lm0, lm1, ld0, ld1, li0, li1, lj0, lj1,
                     am0, am1, ad0, ad1):
    cid = lax.axis_index("c")
    sid = lax.axis_index("s")
    wid = sid * NC + cid
    base_w = wid * SCPW

    zeros16 = jnp.zeros((16,), jnp.float32)
    for r in range(ZR):
        for c in range(8):
            z_v[r, pl.ds(c * 16, 16)] = zeros16

    def zero_body(k, carry):
        pltpu.sync_copy(z_v, acc_sh.at[pl.ds(sid * RPT + k * ZR, ZR)])
        return carry

    lax.fori_loop(0, RPT // ZR, zero_body, 0)
    plsc.subcore_barrier()

    vms = (vm0, vm1)
    vds = (vd0, vd1)
    lmsem = (lm0, lm1)
    ldsem = (ld0, ld1)
    lisem = (li0, li1)
    ljsem = (lj0, lj1)
    amsem = (am0, am1)
    adsem = (ad0, ad1)

    def lm_copy(j, b):
        return pltpu.make_async_copy(
            vm_hbm.at[pl.ds((base_w + j) * SCH, SCH)], vms[b], lmsem[b])

    def ld_copy(j, b):
        return pltpu.make_async_copy(
            vd_hbm.at[pl.ds((base_w + j) * SCH, SCH)], vds[b], ldsem[b])

    def li_copy(j, b):
        return pltpu.make_async_copy(idx_hbm.at[base_w + j], idx_pp.at[b],
                                     lisem[b])

    def lj_copy(j, b):
        return pltpu.make_async_copy(idx2_hbm.at[base_w + j], idx2_pp.at[b],
                                     ljsem[b])

    def am_copy(j, b):
        return pltpu.make_async_copy(vms[b], acc_sh.at[idx_pp.at[b]],
                                     amsem[b])

    def ad_copy(j, b):
        return pltpu.make_async_copy(vds[b], acc_sh.at[idx2_pp.at[b]],
                                     adsem[b])

    def loads_start(j, b):
        lm_copy(j, b).start()
        ld_copy(j, b).start()
        li_copy(j, b).start()
        lj_copy(j, b).start()

    def step(j, b):
        lm_copy(j, b).wait()
        ld_copy(j, b).wait()
        li_copy(j, b).wait()
        lj_copy(j, b).wait()
        am_copy(j, b).start(add=True)
        ad_copy(j, b).start(add=True)
        am_copy(j - 1, 1 - b).wait()
        ad_copy(j - 1, 1 - b).wait()
        loads_start(j + 1, 1 - b)

    loads_start(0, 0)
    lm_copy(0, 0).wait()
    ld_copy(0, 0).wait()
    li_copy(0, 0).wait()
    lj_copy(0, 0).wait()
    am_copy(0, 0).start(add=True)
    ad_copy(0, 0).start(add=True)
    loads_start(1, 1)

    def group(gi, carry):
        j0 = 1 + 2 * gi
        step(j0, 1)
        step(j0 + 1, 0)
        return carry

    lax.fori_loop(0, (SCPW - 2) // 2, group, 0)

    lm_copy(SCPW - 1, 1).wait()
    ld_copy(SCPW - 1, 1).wait()
    li_copy(SCPW - 1, 1).wait()
    lj_copy(SCPW - 1, 1).wait()
    am_copy(SCPW - 1, 1).start(add=True)
    ad_copy(SCPW - 1, 1).start(add=True)
    am_copy(SCPW - 2, 0).wait()
    ad_copy(SCPW - 2, 0).wait()
    am_copy(SCPW - 1, 1).wait()
    ad_copy(SCPW - 1, 1).wait()
    plsc.subcore_barrier()

    def wb_body(k, carry):
        row = sid * RPT + k * WB
        pltpu.sync_copy(acc_sh.at[pl.ds(row, WB)], vm0.at[pl.ds(0, WB)])
        pltpu.sync_copy(vm0.at[pl.ds(0, WB)], out_hbm.at[cid].at[pl.ds(row, WB)])
        return carry

    lax.fori_loop(0, RPT // WB, wb_body, 0)


def _dense1_body(x_ref, wcat_ref, u_ref, bcat_ref, gq_ref, r_ref):
    xb = x_ref[...]
    ti = jnp.clip(xb[:, 0:1].astype(jnp.int32), 0, 1).astype(jnp.float32)
    gqr = jnp.dot(xb, wcat_ref[...], preferred_element_type=jnp.float32)
    usel = u_ref[0:1, :] + ti * (u_ref[1:2, :] - u_ref[0:1, :])
    gqr = gqr + usel + bcat_ref[...]
    gq_ref[...] = gqr[:, :256]
    r_ref[...] = gqr[:, 256:]


def _edge_mlp_body(gq_ref, ea_ref, dmod_ref, wec_ref, wh1_ref, bh1_ref,
                   wh2_ref, bh2_ref, w2_ref, b2_ref, vm_ref, vd_ref):
    gq = gq_ref[...]
    eaa = jnp.dot(ea_ref[...], wec_ref[...], preferred_element_type=jnp.float32)
    g = gq[:, :128] + eaa[:, :128]
    m = gq[:, 128:] + eaa[:, 128:]
    t = jnp.tanh(g)
    t = jnp.tanh(jnp.dot(t, wh1_ref[...], preferred_element_type=jnp.float32)
                 + bh1_ref[...])
    t = jnp.tanh(jnp.dot(t, wh2_ref[...], preferred_element_type=jnp.float32)
                 + bh2_ref[...])
    gate = jnp.sum(t * w2_ref[...], axis=1, keepdims=True) + b2_ref[...]
    ex = jnp.exp(gate)
    vm_ref[...] = ex * m
    onehot = (lax.broadcasted_iota(jnp.int32, (1, 128), 1) == dmod_ref[...])
    vd_ref[...] = ex * onehot.astype(jnp.float32)


def _combine2_body(p_ref, den_ref, r1_ref, bl1_ref, wcat2_ref, bcat2_ref,
                   gq2_ref, r2_ref):
    s = p_ref[0] + p_ref[1]
    aggr = s / (den_ref[...] + 1e-16)
    h2 = jnp.maximum(aggr + bl1_ref[...] + r1_ref[...], 0.0)
    gqr = jnp.dot(h2, wcat2_ref[...], preferred_element_type=jnp.float32)
    gqr = gqr + bcat2_ref[...]
    gq2_ref[...] = gqr[:, :256]
    r2_ref[...] = gqr[:, 256:]


def _final_body(p_ref, den_ref, r2_ref, bl2_ref, o_ref):
    s = p_ref[0] + p_ref[1]
    aggr = s / (den_ref[...] + 1e-16)
    o_ref[...] = aggr + bl2_ref[...] + r2_ref[...]


_NB = 1000
_EB = 640


def _rows_spec(blk, width):
    return pl.BlockSpec((blk, width), lambda i: (i, 0))


def _bcast_spec(shape):
    return pl.BlockSpec(shape, lambda i: tuple(0 for _ in shape))


def _dense(x_in, wcat, u, bcat):
    return pl.pallas_call(
        _dense1_body,
        grid=(N // _NB,),
        in_specs=[_rows_spec(_NB, 128), _bcast_spec((128, 384)),
                  _bcast_spec((2, 384)), _bcast_spec((1, 384))],
        out_specs=[_rows_spec(_NB, 256), _rows_spec(_NB, 128)],
        out_shape=[jax.ShapeDtypeStruct((N, 256), jnp.float32),
                   jax.ShapeDtypeStruct((N, 128), jnp.float32)],
    )(x_in, wcat, u, bcat)


def _edge_mlp(gqg, eap, dmod, wec, wh1, bh1, wh2, bh2, w2, b2):
    return pl.pallas_call(
        _edge_mlp_body,
        grid=(E_PAD // _EB,),
        in_specs=[_rows_spec(_EB, 256), _rows_spec(_EB, EDGE_DIM),
                  _rows_spec(_EB, 1),
                  _bcast_spec((EDGE_DIM, 256)), _bcast_spec((128, 128)),
                  _bcast_spec((1, 128)), _bcast_spec((128, 128)),
                  _bcast_spec((1, 128)), _bcast_spec((1, 128)),
                  _bcast_spec((1, 1))],
        out_specs=[_rows_spec(_EB, 128), _rows_spec(_EB, 128)],
        out_shape=[jax.ShapeDtypeStruct((E_PAD, 128), jnp.float32),
                   jax.ShapeDtypeStruct((E_PAD, 128), jnp.float32)],
    )(gqg, eap, dmod, wec, wh1, bh1, wh2, bh2, w2, b2)


def _combine2(p, den, r1, bl1, wcat2, bcat2):
    return pl.pallas_call(
        _combine2_body,
        grid=(N // _NB,),
        in_specs=[pl.BlockSpec((NC, _NB, 128), lambda i: (0, i, 0)),
                  _rows_spec(_NB, 1),
                  _rows_spec(_NB, 128), _bcast_spec((1, 128)),
                  _bcast_spec((128, 384)), _bcast_spec((1, 384))],
        out_specs=[_rows_spec(_NB, 256), _rows_spec(_NB, 128)],
        out_shape=[jax.ShapeDtypeStruct((N, 256), jnp.float32),
                   jax.ShapeDtypeStruct((N, 128), jnp.float32)],
    )(p, den, r1, bl1, wcat2, bcat2)


def _final(p, den, r2, bl2):
    return pl.pallas_call(
        _final_body,
        grid=(N // _NB,),
        in_specs=[pl.BlockSpec((NC, _NB, 128), lambda i: (0, i, 0)),
                  _rows_spec(_NB, 1),
                  _rows_spec(_NB, 128), _bcast_spec((1, 128))],
        out_specs=_rows_spec(_NB, 128),
        out_shape=jax.ShapeDtypeStruct((N, 128), jnp.float32),
    )(p, den, r2, bl2)


def kernel(x, edge_index, edge_attr, params):
    p = params
    f32 = jnp.float32

    zrow = jnp.zeros((1, 128), f32)
    W1, Wl, Wr = p['l1_gate_W1'], p['l1_Wl'], p['l1_Wr']
    wcat1 = jnp.concatenate([
        jnp.concatenate([zrow, W1[:127]], 0),
        jnp.concatenate([zrow, Wl[:127]], 0),
        jnp.concatenate([zrow, Wr[:127]], 0)], 1)
    u1 = jnp.concatenate([p['embed'] @ W1[127:129],
                          p['embed'] @ Wl[127:129],
                          p['embed'] @ Wr[127:129]], 1)
    bcat1 = jnp.concatenate([p['l1_gate_b1'], jnp.zeros((256,), f32)])[None]
    wec1 = jnp.concatenate([W1[129:], Wl[129:]], 1)

    z16 = jnp.zeros((16, 128), f32)
    W1_2, Wl_2, Wr_2 = p['l2_gate_W1'], p['l2_Wl'], p['l2_Wr']
    wcat2 = jnp.concatenate([
        jnp.concatenate([W1_2[:112], z16], 0),
        jnp.concatenate([Wl_2[:112], z16], 0),
        Wr_2], 1)
    bcat2 = jnp.concatenate([p['l2_gate_b1'], jnp.zeros((256,), f32)])[None]
    wec2 = jnp.concatenate([W1_2[112:], Wl_2[112:]], 1)

    src = edge_index[0]
    dst = edge_index[1]
    pad = E_PAD - E
    srcp = jnp.concatenate([src, jnp.zeros((pad,), jnp.int32)])
    srcp2d = srcp.reshape(E_PAD // CH, CH)
    dstp = jnp.concatenate([dst, jnp.full((pad,), N, jnp.int32)])
    dstp2d = dstp.reshape(E_PAD // SCH, SCH)
    dst2p2d = (DB + dstp // 128).reshape(E_PAD // SCH, SCH)
    dmod = (dstp % 128).reshape(E_PAD, 1)
    eap = jnp.concatenate([edge_attr, jnp.zeros((pad, EDGE_DIM), f32)], 0)

    def layer(gq, lname, wec):
        gqg = _sc_gather(gq, srcp2d)
        vm, vd = _edge_mlp(gqg, eap, dmod, wec,
                           p[lname + '_gate_Wh1'], p[lname + '_gate_bh1'][None],
                           p[lname + '_gate_Wh2'], p[lname + '_gate_bh2'][None],
                           p[lname + '_gate_W2'].T, p[lname + '_gate_b2'][None])
        acc = _sc_scatter(vm, vd, dstp2d, dst2p2d)
        den = acc[:, DB:DB + (N + 127) // 128, :].sum(0).reshape(-1)[:N, None]
        return acc, den

    gq1, r1 = _dense(x, wcat1, u1, bcat1)
    p1, den1 = layer(gq1, 'l1', wec1)
    gq2, r2 = _combine2(p1, den1, r1, p['l1_bl'][None], wcat2, bcat2)
    p2, den2 = layer(gq2, 'l2', wec2)
    return _final(p2, den2, r2, p['l2_bl'][None])


# speedup vs baseline: 3.7304x; 1.0256x over previous
import functools

import jax
import jax.numpy as jnp
from jax import lax
from jax.experimental import pallas as pl
from jax.experimental.pallas import tpu as pltpu
from jax.experimental.pallas import tpu_sc as plsc

N = 10000
E = 160000
EDGE_DIM = 16
NP = 10240
DB = 10008
NT = 10240
E_PAD = 163840
NC, NS = 2, 16
NW = NC * NS
CH = 128
CPW = E_PAD // (NW * CH)
SCH = 64
SCPW = E_PAD // (NW * SCH)
RPT = NT // NS
WB = 64
ZR = 8


def _mesh():
    return plsc.VectorSubcoreMesh(
        core_axis_name="c", subcore_axis_name="s",
        num_cores=NC, num_subcores=NS)


@functools.cache
def _sc_gather_kernel():
    return pl.kernel(
        _sc_gather_body,
        out_type=jax.ShapeDtypeStruct((E_PAD, 256), jnp.float32),
        mesh=_mesh(),
        scratch_types=[
            pltpu.VMEM((CPW, CH), jnp.int32),
            pltpu.VMEM((CH, 256), jnp.float32),
            pltpu.VMEM((CH, 256), jnp.float32),
            pltpu.VMEM((CH, 256), jnp.float32),
            pltpu.SemaphoreType.DMA,
            pltpu.SemaphoreType.DMA,
            pltpu.SemaphoreType.DMA,
            pltpu.SemaphoreType.DMA,
            pltpu.SemaphoreType.DMA,
            pltpu.SemaphoreType.DMA,
        ],
    )


def _sc_gather(table, idx2d):
    return _sc_gather_kernel()(table, idx2d)


def _sc_gather_body(table_hbm, idx_hbm, out_hbm, idx_all, r0, r1, r2,
                    g0, g1, g2, s0, s1, s2):
    wid = lax.axis_index("s") * NC + lax.axis_index("c")
    base_w = wid * CPW
    pltpu.sync_copy(idx_hbm.at[pl.ds(base_w, CPW)], idx_all)
    rows = (r0, r1, r2)
    gsem = (g0, g1, g2)
    ssem = (s0, s1, s2)

    def g_copy(j, b):
        return pltpu.make_async_copy(table_hbm.at[idx_all.at[j]], rows[b],
                                     gsem[b])

    def s_copy(j, b):
        return pltpu.make_async_copy(
            rows[b], out_hbm.at[pl.ds((base_w + j) * CH, CH)], ssem[b])

    g_copy(0, 0).start()
    g_copy(1, 1).start()
    g_copy(0, 0).wait()
    s_copy(0, 0).start()
    g_copy(2, 2).start()

    def group(gi, carry):
        j0 = 1 + 3 * gi
        g_copy(j0, 1).wait()
        s_copy(j0, 1).start()
        s_copy(j0 - 1, 0).wait()
        g_copy(j0 + 2, 0).start()
        g_copy(j0 + 1, 2).wait()
        s_copy(j0 + 1, 2).start()
        s_copy(j0, 1).wait()
        g_copy(j0 + 3, 1).start()
        g_copy(j0 + 2, 0).wait()
        s_copy(j0 + 2, 0).start()
        s_copy(j0 + 1, 2).wait()
        g_copy(j0 + 4, 2).start()
        return carry

    lax.fori_loop(0, (CPW - 4) // 3, group, 0)

    g_copy(CPW - 3, 1).wait()
    s_copy(CPW - 3, 1).start()
    s_copy(CPW - 4, 0).wait()
    g_copy(CPW - 1, 0).start()
    g_copy(CPW - 2, 2).wait()
    s_copy(CPW - 2, 2).start()
    g_copy(CPW - 1, 0).wait()
    s_copy(CPW - 1, 0).start()
    s_copy(CPW - 3, 1).wait()
    s_copy(CPW - 2, 2).wait()
    s_copy(CPW - 1, 0).wait()


@functools.cache
def _sc_scatter_kernel():
    return pl.kernel(
        _sc_scatter_body,
        out_type=jax.ShapeDtypeStruct((NC, NT, 128), jnp.float32),
        mesh=_mesh(),
        scratch_types=[
            pltpu.VMEM((2, SCH), jnp.int32),
            pltpu.VMEM((2, SCH), jnp.int32),
            pltpu.VMEM((SCH, 128), jnp.float32),
            pltpu.VMEM((SCH, 128), jnp.float32),
            pltpu.VMEM((SCH, 128), jnp.float32),
            pltpu.VMEM((SCH, 128), jnp.float32),
            pltpu.VMEM((ZR, 128), jnp.float32),
            pltpu.VMEM_SHARED((NT, 128), jnp.float32),
            pltpu.SemaphoreType.DMA,
            pltpu.SemaphoreType.DMA,
            pltpu.SemaphoreType.DMA,
            pltpu.SemaphoreType.DMA,
            pltpu.SemaphoreType.DMA,
            pltpu.SemaphoreType.DMA,
            pltpu.SemaphoreType.DMA,
            pltpu.SemaphoreType.DMA,
            pltpu.SemaphoreType.DMA,
            pltpu.SemaphoreType.DMA,
            pltpu.SemaphoreType.DMA,
            pltpu.SemaphoreType.DMA,
        ],
    )


def _sc_scatter(vm, vd, idx2d, idx2_2d):
    return _sc_scatter_kernel()(vm, vd, idx2d, idx2_2d)


def _sc_scatter_body(vm_hbm, vd_hbm, idx_hbm, idx2_hbm, out_hbm,
                     idx_pp, idx2_pp, vm0, vm1, vd0, vd1,
                     z_v, acc_sh,
                     lm0, lm1, ld0, ld1, li0, li1, lj0, lj1,
                     am0, am1, ad0, ad1):
    cid = lax.axis_index("c")
    sid = lax.axis_index("s")
    wid = sid * NC + cid
    base_w = wid * SCPW

    zeros16 = jnp.zeros((16,), jnp.float32)
    for r in range(ZR):
        for c in range(8):
            z_v[r, pl.ds(c * 16, 16)] = zeros16

    def z_copy(k):
        return pltpu.make_async_copy(
            z_v, acc_sh.at[pl.ds(sid * RPT + k * ZR, ZR)], ld0)

    def zero_start(k, carry):
        z_copy(k).start()
        return carry

    def zero_drain(k, carry):
        z_copy(k).wait()
        return carry

    lax.fori_loop(0, RPT // ZR, zero_start, 0)
    lax.fori_loop(0, RPT // ZR, zero_drain, 0)
    plsc.subcore_barrier()

    vms = (vm0, vm1)
    vds = (vd0, vd1)
    lmsem = (lm0, lm1)
    ldsem = (ld0, ld1)
    lisem = (li0, li1)
    ljsem = (lj0, lj1)
    amsem = (am0, am1)
    adsem = (ad0, ad1)

    def lm_copy(j, b):
        return pltpu.make_async_copy(
            vm_hbm.at[pl.ds((base_w + j) * SCH, SCH)], vms[b], lmsem[b])

    def ld_copy(j, b):
        return pltpu.make_async_copy(
            vd_hbm.at[pl.ds((base_w + j) * SCH, SCH)], vds[b], ldsem[b])

    def li_copy(j, b):
        return pltpu.make_async_copy(idx_hbm.at[base_w + j], idx_pp.at[b],
                                     lisem[b])

    def lj_copy(j, b):
        return pltpu.make_async_copy(idx2_hbm.at[base_w + j], idx2_pp.at[b],
                                     ljsem[b])

    def am_copy(j, b):
        return pltpu.make_async_copy(vms[b], acc_sh.at[idx_pp.at[b]],
                                     amsem[b])

    def ad_copy(j, b):
        return pltpu.make_async_copy(vds[b], acc_sh.at[idx2_pp.at[b]],
                                     adsem[b])

    def loads_start(j, b):
        lm_copy(j, b).start()
        ld_copy(j, b).start()
        li_copy(j, b).start()
        lj_copy(j, b).start()

    def step(j, b):
        lm_copy(j, b).wait()
        ld_copy(j, b).wait()
        li_copy(j, b).wait()
        lj_copy(j, b).wait()
        am_copy(j, b).start(add=True)
        ad_copy(j, b).start(add=True)
        am_copy(j - 1, 1 - b).wait()
        ad_copy(j - 1, 1 - b).wait()
        loads_start(j + 1, 1 - b)

    loads_start(0, 0)
    lm_copy(0, 0).wait()
    ld_copy(0, 0).wait()
    li_copy(0, 0).wait()
    lj_copy(0, 0).wait()
    am_copy(0, 0).start(add=True)
    ad_copy(0, 0).start(add=True)
    loads_start(1, 1)

    def group(gi, carry):
        j0 = 1 + 2 * gi
        step(j0, 1)
        step(j0 + 1, 0)
        return carry

    lax.fori_loop(0, (SCPW - 2) // 2, group, 0)

    lm_copy(SCPW - 1, 1).wait()
    ld_copy(SCPW - 1, 1).wait()
    li_copy(SCPW - 1, 1).wait()
    lj_copy(SCPW - 1, 1).wait()
    am_copy(SCPW - 1, 1).start(add=True)
    ad_copy(SCPW - 1, 1).start(add=True)
    am_copy(SCPW - 2, 0).wait()
    ad_copy(SCPW - 2, 0).wait()
    am_copy(SCPW - 1, 1).wait()
    ad_copy(SCPW - 1, 1).wait()
    plsc.subcore_barrier()

    def wr_copy(k, b):
        return pltpu.make_async_copy(
            acc_sh.at[pl.ds(sid * RPT + k * WB, WB)], vms[b], lmsem[b])

    def ww_copy(k, b):
        return pltpu.make_async_copy(
            vms[b], out_hbm.at[cid].at[pl.ds(sid * RPT + k * WB, WB)],
            amsem[b])

    nwb = RPT // WB
    wr_copy(0, 0).start()
    for k in range(nwb):
        b = k % 2
        wr_copy(k, b).wait()
        ww_copy(k, b).start()
        if k + 1 < nwb:
            if k >= 1:
                ww_copy(k - 1, 1 - b).wait()
            wr_copy(k + 1, 1 - b).start()
    ww_copy(nwb - 2, 0).wait()
    ww_copy(nwb - 1, 1).wait()


def _dense1_body(x_ref, wcat_ref, u_ref, bcat_ref, gq_ref, r_ref):
    xb = x_ref[...]
    ti = jnp.clip(xb[:, 0:1].astype(jnp.int32), 0, 1).astype(jnp.float32)
    gqr = jnp.dot(xb, wcat_ref[...], preferred_element_type=jnp.float32)
    usel = u_ref[0:1, :] + ti * (u_ref[1:2, :] - u_ref[0:1, :])
    gqr = gqr + usel + bcat_ref[...]
    gq_ref[...] = gqr[:, :256]
    r_ref[...] = gqr[:, 256:]


def _edge_mlp_body(gq_ref, ea_ref, dmod_ref, wec_ref, wh1_ref, bh1_ref,
                   wh2_ref, bh2_ref, w2_ref, b2_ref, vm_ref, vd_ref):
    gq = gq_ref[...]
    eaa = jnp.dot(ea_ref[...], wec_ref[...], preferred_element_type=jnp.float32)
    g = gq[:, :128] + eaa[:, :128]
    m = gq[:, 128:] + eaa[:, 128:]
    t = jnp.tanh(g)
    t = jnp.tanh(jnp.dot(t, wh1_ref[...], preferred_element_type=jnp.float32)
                 + bh1_ref[...])
    t = jnp.tanh(jnp.dot(t, wh2_ref[...], preferred_element_type=jnp.float32)
                 + bh2_ref[...])
    gate = jnp.sum(t * w2_ref[...], axis=1, keepdims=True) + b2_ref[...]
    ex = jnp.exp(gate)
    vm_ref[...] = ex * m
    onehot = (lax.broadcasted_iota(jnp.int32, (1, 128), 1) == dmod_ref[...])
    vd_ref[...] = ex * onehot.astype(jnp.float32)


def _combine2_body(p_ref, den_ref, r1_ref, bl1_ref, wcat2_ref, bcat2_ref,
                   gq2_ref, r2_ref):
    s = p_ref[0] + p_ref[1]
    aggr = s / (den_ref[...] + 1e-16)
    h2 = jnp.maximum(aggr + bl1_ref[...] + r1_ref[...], 0.0)
    gqr = jnp.dot(h2, wcat2_ref[...], preferred_element_type=jnp.float32)
    gqr = gqr + bcat2_ref[...]
    gq2_ref[...] = gqr[:, :256]
    r2_ref[...] = gqr[:, 256:]


def _final_body(p_ref, den_ref, r2_ref, bl2_ref, o_ref):
    s = p_ref[0] + p_ref[1]
    aggr = s / (den_ref[...] + 1e-16)
    o_ref[...] = aggr + bl2_ref[...] + r2_ref[...]


_NB = 1000
_EB = 640


def _rows_spec(blk, width):
    return pl.BlockSpec((blk, width), lambda i: (i, 0))


def _bcast_spec(shape):
    return pl.BlockSpec(shape, lambda i: tuple(0 for _ in shape))


def _dense(x_in, wcat, u, bcat):
    return pl.pallas_call(
        _dense1_body,
        grid=(N // _NB,),
        in_specs=[_rows_spec(_NB, 128), _bcast_spec((128, 384)),
                  _bcast_spec((2, 384)), _bcast_spec((1, 384))],
        out_specs=[_rows_spec(_NB, 256), _rows_spec(_NB, 128)],
        out_shape=[jax.ShapeDtypeStruct((N, 256), jnp.float32),
                   jax.ShapeDtypeStruct((N, 128), jnp.float32)],
    )(x_in, wcat, u, bcat)


def _edge_mlp(gqg, eap, dmod, wec, wh1, bh1, wh2, bh2, w2, b2):
    return pl.pallas_call(
        _edge_mlp_body,
        grid=(E_PAD // _EB,),
        in_specs=[_rows_spec(_EB, 256), _rows_spec(_EB, EDGE_DIM),
                  _rows_spec(_EB, 1),
                  _bcast_spec((EDGE_DIM, 256)), _bcast_spec((128, 128)),
                  _bcast_spec((1, 128)), _bcast_spec((128, 128)),
                  _bcast_spec((1, 128)), _bcast_spec((1, 128)),
                  _bcast_spec((1, 1))],
        out_specs=[_rows_spec(_EB, 128), _rows_spec(_EB, 128)],
        out_shape=[jax.ShapeDtypeStruct((E_PAD, 128), jnp.float32),
                   jax.ShapeDtypeStruct((E_PAD, 128), jnp.float32)],
    )(gqg, eap, dmod, wec, wh1, bh1, wh2, bh2, w2, b2)


def _combine2(p, den, r1, bl1, wcat2, bcat2):
    return pl.pallas_call(
        _combine2_body,
        grid=(N // _NB,),
        in_specs=[pl.BlockSpec((NC, _NB, 128), lambda i: (0, i, 0)),
                  _rows_spec(_NB, 1),
                  _rows_spec(_NB, 128), _bcast_spec((1, 128)),
                  _bcast_spec((128, 384)), _bcast_spec((1, 384))],
        out_specs=[_rows_spec(_NB, 256), _rows_spec(_NB, 128)],
        out_shape=[jax.ShapeDtypeStruct((N, 256), jnp.float32),
                   jax.ShapeDtypeStruct((N, 128), jnp.float32)],
    )(p, den, r1, bl1, wcat2, bcat2)


def _final(p, den, r2, bl2):
    return pl.pallas_call(
        _final_body,
        grid=(N // _NB,),
        in_specs=[pl.BlockSpec((NC, _NB, 128), lambda i: (0, i, 0)),
                  _rows_spec(_NB, 1),
                  _rows_spec(_NB, 128), _bcast_spec((1, 128))],
        out_specs=_rows_spec(_NB, 128),
        out_shape=jax.ShapeDtypeStruct((N, 128), jnp.float32),
    )(p, den, r2, bl2)


def kernel(x, edge_index, edge_attr, params):
    p = params
    f32 = jnp.float32

    zrow = jnp.zeros((1, 128), f32)
    W1, Wl, Wr = p['l1_gate_W1'], p['l1_Wl'], p['l1_Wr']
    wcat1 = jnp.concatenate([
        jnp.concatenate([zrow, W1[:127]], 0),
        jnp.concatenate([zrow, Wl[:127]], 0),
        jnp.concatenate([zrow, Wr[:127]], 0)], 1)
    u1 = jnp.concatenate([p['embed'] @ W1[127:129],
                          p['embed'] @ Wl[127:129],
                          p['embed'] @ Wr[127:129]], 1)
    bcat1 = jnp.concatenate([p['l1_gate_b1'], jnp.zeros((256,), f32)])[None]
    wec1 = jnp.concatenate([W1[129:], Wl[129:]], 1)

    z16 = jnp.zeros((16, 128), f32)
    W1_2, Wl_2, Wr_2 = p['l2_gate_W1'], p['l2_Wl'], p['l2_Wr']
    wcat2 = jnp.concatenate([
        jnp.concatenate([W1_2[:112], z16], 0),
        jnp.concatenate([Wl_2[:112], z16], 0),
        Wr_2], 1)
    bcat2 = jnp.concatenate([p['l2_gate_b1'], jnp.zeros((256,), f32)])[None]
    wec2 = jnp.concatenate([W1_2[112:], Wl_2[112:]], 1)

    src = edge_index[0]
    dst = edge_index[1]
    pad = E_PAD - E
    srcp = jnp.concatenate([src, jnp.zeros((pad,), jnp.int32)])
    srcp2d = srcp.reshape(E_PAD // CH, CH)
    dstp = jnp.concatenate([dst, jnp.full((pad,), N, jnp.int32)])
    dstp2d = dstp.reshape(E_PAD // SCH, SCH)
    dst2p2d = (DB + dstp // 128).reshape(E_PAD // SCH, SCH)
    dmod = (dstp % 128).reshape(E_PAD, 1)
    eap = jnp.concatenate([edge_attr, jnp.zeros((pad, EDGE_DIM), f32)], 0)

    def layer(gq, lname, wec):
        gqg = _sc_gather(gq, srcp2d)
        vm, vd = _edge_mlp(gqg, eap, dmod, wec,
                           p[lname + '_gate_Wh1'], p[lname + '_gate_bh1'][None],
                           p[lname + '_gate_Wh2'], p[lname + '_gate_bh2'][None],
                           p[lname + '_gate_W2'].T, p[lname + '_gate_b2'][None])
        acc = _sc_scatter(vm, vd, dstp2d, dst2p2d)
        den = acc[:, DB:DB + (N + 127) // 128, :].sum(0).reshape(-1)[:N, None]
        return acc, den

    gq1, r1 = _dense(x, wcat1, u1, bcat1)
    p1, den1 = layer(gq1, 'l1', wec1)
    gq2, r2 = _combine2(p1, den1, r1, p['l1_bl'][None], wcat2, bcat2)
    p2, den2 = layer(gq2, 'l2', wec2)
    return _final(p2, den2, r2, p['l2_bl'][None])
